# baseline (speedup 1.0000x reference)
import jax
import jax.numpy as jnp
from jax.experimental import pallas as pl

_N = 32
_R = 3 * 512
_W = 512
_HB = _R // 2
_G = 2 * _N


def _shift_body(x_ref, t_ref, o_ref):
    i = pl.program_id(0)

    @pl.when(i < _G - 2)
    def _():
        o_ref[...] = t_ref[...]

    @pl.when(i >= _G - 2)
    def _():
        o_ref[...] = x_ref[...]


def kernel(x, tensors):
    x2 = x.reshape(_R, _W)
    t2 = tensors.reshape(_N * _R, _W)
    out = pl.pallas_call(
        _shift_body,
        grid=(_G,),
        in_specs=[
            pl.BlockSpec((_HB, _W), lambda i: (jnp.maximum(i - (_G - 2), 0), 0)),
            pl.BlockSpec((_HB, _W), lambda i: (jnp.minimum(i + 2, _G - 1), 0)),
        ],
        out_specs=pl.BlockSpec((_HB, _W), lambda i: (i, 0)),
        out_shape=jax.ShapeDtypeStruct((_N * _R, _W), jnp.float32),
    )(x2, t2)
    return out.reshape(tensors.shape)


# speedup vs baseline: 1.2053x; 1.2053x over previous
import jax
import jax.numpy as jnp
from jax.experimental import pallas as pl

_N = 32
_R = 3 * 512
_W = 512


def _shift_body(x_ref, t_ref, o_ref):
    i = pl.program_id(0)

    @pl.when(i < _N - 1)
    def _():
        o_ref[...] = t_ref[...]

    @pl.when(i == _N - 1)
    def _():
        o_ref[...] = x_ref[...]


def kernel(x, tensors):
    x2 = x.reshape(_R, _W)
    t2 = tensors.reshape(_N * _R, _W)
    out = pl.pallas_call(
        _shift_body,
        grid=(_N,),
        in_specs=[
            pl.BlockSpec((_R, _W), lambda i: (0, 0)),
            pl.BlockSpec((_R, _W), lambda i: (jnp.minimum(i + 1, _N - 1), 0)),
        ],
        out_specs=pl.BlockSpec((_R, _W), lambda i: (i, 0)),
        out_shape=jax.ShapeDtypeStruct((_N * _R, _W), jnp.float32),
    )(x2, t2)
    return out.reshape(tensors.shape)
